# baseline (speedup 1.0000x reference)
import jax
import jax.numpy as jnp
from jax.experimental import pallas as pl
from jax.experimental.pallas import tpu as pltpu

_B, _C, _H, _W = 2, 128, 256, 256
_PS = 16
_NH = _H // _PS
_NW = _W // _PS
_N = _NH * _NW
_HEADS = 8
_OUTC = _C // _HEADS
_THR = 0.5
_DT = 128
_NT = _N // _DT
_RB = 64
_NRB = _H // _RB
_RPP = _RB // _PS
_F32 = jnp.float32
_NS = 20


def _layer_tile(xin, neg_full, t, wl, blc, blr, wr, brwe, attr, bias):
    xlT = jax.lax.dot_general(wl, xin, (((0,), (1,)), ((), ())),
                              preferred_element_type=_F32) + blc
    xl = jnp.dot(xin, wl, preferred_element_type=_F32) + blr
    h_ids = jax.lax.broadcasted_iota(jnp.int32, (_HEADS, _C), 0)
    c_ids = jax.lax.broadcasted_iota(jnp.int32, (_HEADS, _C), 1) // _OUTC
    attmT = jnp.where(h_ids == c_ids, attr, 0.0)
    attmB = jnp.broadcast_to(attmT[None], (_DT, _HEADS, _C))
    sel = jnp.where(h_ids == c_ids, 1.0, 0.0).astype(_F32)
    alin = jax.lax.dot_general(attmT, xlT, (((1,), (0,)), ((), ())),
                               preferred_element_type=_F32)
    xre = (jnp.dot(xin[t * _DT:(t + 1) * _DT], wr,
                   preferred_element_type=_F32) + brwe)
    pairT = xlT[None, :, :] + xre[:, :, None]
    absT = jnp.abs(pairT)
    habs = jax.lax.dot_general(attmB, absT, (((2,), (1,)), ((0,), (0,))),
                               preferred_element_type=_F32)
    are = jax.lax.dot_general(xre, attmT, (((1,), (1,)), ((), ())),
                              preferred_element_type=_F32)
    neg = neg_full[t * _DT:(t + 1) * _DT]
    logits = (0.6 * (alin[None, :, :] + are[:, :, None])
              + 0.4 * habs + neg[:, None, :])
    m = jnp.max(logits, axis=2, keepdims=True)
    p = jnp.exp(logits - m)
    alpha = p / jnp.sum(p, axis=2, keepdims=True)
    agg = jnp.dot(alpha.reshape(_DT * _HEADS, _N), xl,
                  preferred_element_type=_F32).reshape(_DT, _HEADS, _C)
    out = jnp.sum(agg * sel[None], axis=1) + bias
    return jnp.where(out > 0, out, jnp.exp(out) - 1.0)


def _mega_body(x_ref,
               wl1_ref, blc1_ref, blr1_ref, wr1_ref, brwe1_ref, attr1_ref,
               bias1_ref,
               wl2_ref, blc2_ref, blr2_ref, wr2_ref, brwe2_ref, attr2_ref,
               bias2_ref, o_ref, nf_s, neg_s, h1_s, g_s):
    s = pl.program_id(0)
    is_g = ((s == 5) | (s == 7) | (s == 11) | (s == 13))
    is_pool = (s < 4) | (s == 4) | (s == 6) | (s == 8) | (s == 9)
    gsub = jnp.where((s == 5) | (s == 11), 0, 1)
    gb = jnp.where(s < 10, 0, 1)

    @pl.when(is_pool)
    def _pool():
        bsel = jnp.where(s < 4, 0, 1)
        rb = jnp.where(s < 4, s, jnp.where(s < 8, (s - 4) // 2, s - 6))
        xb = x_ref[0]
        w_ids = jax.lax.broadcasted_iota(jnp.int32, (_W, _NW), 0) // _PS
        p_ids = jax.lax.broadcasted_iota(jnp.int32, (_W, _NW), 1)
        pmat = jnp.where(w_ids == p_ids, 1.0 / (_PS * _PS), 0.0).astype(_F32)
        for r in range(_RPP):
            srow = jnp.sum(xb[:, r * _PS:(r + 1) * _PS, :], axis=1)
            rows = jax.lax.dot_general(pmat, srow, (((0,), (1,)), ((), ())),
                                       preferred_element_type=_F32)
            nf_s[bsel, pl.ds(rb * (_RPP * _NW) + r * _NW, _NW)] = rows

    @pl.when(jnp.logical_and(is_g, gsub == 0))
    def _g0():
        nf = nf_s[gb]
        mu = jnp.mean(nf, axis=-1, keepdims=True)
        xc = nf - mu
        num = jax.lax.dot_general(xc, xc, (((1,), (1,)), ((), ())),
                                  preferred_element_type=_F32)
        nrm = jnp.sqrt(jnp.sum(xc * xc, axis=-1, keepdims=True))
        outer = jax.lax.dot_general(nrm, nrm, (((1,), (1,)), ((), ())),
                                    preferred_element_type=_F32)
        corr = num / (outer + 1e-8)
        neg_full = jnp.where(corr > _THR, 0.0, -1e30).astype(_F32)
        neg_s[...] = neg_full
        for t in range(_NT):
            h1_s[pl.ds(t * _DT, _DT)] = _layer_tile(
                nf, neg_full, t, wl1_ref[...], blc1_ref[...], blr1_ref[...],
                wr1_ref[...], brwe1_ref[...], attr1_ref[...], bias1_ref[...])

    @pl.when(jnp.logical_and(is_g, gsub == 1))
    def _g1():
        h1 = h1_s[...]
        neg_full = neg_s[...]
        for t in range(_NT):
            h2t = _layer_tile(
                h1, neg_full, t, wl2_ref[...], blc2_ref[...], blr2_ref[...],
                wr2_ref[...], brwe2_ref[...], attr2_ref[...], bias2_ref[...])
            for ph in range(_DT // _NW):
                g_s[gb, t * (_DT // _NW) + ph] = (
                    h2t[ph * _NW:(ph + 1) * _NW].T)

    @pl.when(jnp.logical_and(jnp.logical_not(is_pool),
                             jnp.logical_not(is_g)))
    def _add():
        bsel = jnp.where(s < 16, 0, 1)
        rb = jnp.where(s < 14, (s - 10) // 2, jnp.where(s < 16, s - 12,
                                                        s - 16))
        xb = x_ref[0]
        p_ids = jax.lax.broadcasted_iota(jnp.int32, (_NW, _W), 0)
        w_ids = jax.lax.broadcasted_iota(jnp.int32, (_NW, _W), 1) // _PS
        emat = jnp.where(p_ids == w_ids, 1.0, 0.0).astype(_F32)
        for r in range(_RPP):
            gr = g_s[bsel, pl.ds(rb * _RPP + r, 1)][0]
            wide = jnp.dot(gr, emat, preferred_element_type=_F32)
            o_ref[0, :, r * _PS:(r + 1) * _PS, :] = (
                xb[:, r * _PS:(r + 1) * _PS, :] + wide[:, None, :])


def _x_index(s):
    b = jnp.where(s < 4, 0, jnp.where(s < 10, 1, jnp.where(s < 16, 0, 1)))
    row = jnp.where(s < 4, s,
                    jnp.where(s < 8, (s - 4) // 2,
                              jnp.where(s < 10, s - 6,
                                        jnp.where(s < 14, (s - 10) // 2,
                                                  jnp.where(s < 16, s - 12,
                                                            s - 16)))))
    return (b, 0, row, 0)


def _o_index(s):
    b = jnp.where(s < 16, 0, 1)
    row = jnp.where(s < 12, 0,
                    jnp.where(s < 14, (s - 10) // 2,
                              jnp.where(s < 16, s - 12, s - 16)))
    return (b, 0, row, 0)


def kernel(x, Wl1, bl1, Wr1, br1, We1, att1, bias1,
           Wl2, bl2, Wr2, br2, We2, att2, bias2):
    wspec = pl.BlockSpec((_C, _C), lambda s: (0, 0))
    rspec = pl.BlockSpec((1, _C), lambda s: (0, 0))
    cspec = pl.BlockSpec((_C, 1), lambda s: (0, 0))
    return pl.pallas_call(
        _mega_body,
        grid=(_NS,),
        in_specs=[
            pl.BlockSpec((1, _C, _RB, _W), _x_index),
            wspec, cspec, rspec, wspec, rspec, rspec, rspec,
            wspec, cspec, rspec, wspec, rspec, rspec, rspec,
        ],
        out_specs=pl.BlockSpec((1, _C, _RB, _W), _o_index),
        out_shape=jax.ShapeDtypeStruct((_B, _C, _H, _W), _F32),
        scratch_shapes=[
            pltpu.VMEM((_B, _N, _C), _F32),
            pltpu.VMEM((_N, _N), _F32),
            pltpu.VMEM((_N, _C), _F32),
            pltpu.VMEM((_B, _NH, _C, _NW), _F32),
        ],
    )(x,
      Wl1, bl1.reshape(_C, 1), bl1.reshape(1, _C), Wr1,
      (br1 + We1.reshape(-1)).reshape(1, _C), att1.reshape(1, _C),
      bias1.reshape(1, _C),
      Wl2, bl2.reshape(_C, 1), bl2.reshape(1, _C), Wr2,
      (br2 + We2.reshape(-1)).reshape(1, _C), att2.reshape(1, _C),
      bias2.reshape(1, _C))


# speedup vs baseline: 1.0689x; 1.0689x over previous
import jax
import jax.numpy as jnp
from jax.experimental import pallas as pl
from jax.experimental.pallas import tpu as pltpu

_B, _C, _H, _W = 2, 128, 256, 256
_PS = 16
_NH = _H // _PS
_NW = _W // _PS
_N = _NH * _NW
_HEADS = 8
_OUTC = _C // _HEADS
_THR = 0.5
_DT = 128
_NT = _N // _DT
_RB = 64
_NRB = _H // _RB
_RPP = _RB // _PS
_F32 = jnp.float32
_NS = 6 * _NRB


def _layer_tile(xin, neg_full, t, wl, blc, blr, wr, brwe, attr, bias):
    xlT = jax.lax.dot_general(wl, xin, (((0,), (1,)), ((), ())),
                              preferred_element_type=_F32) + blc
    xl = jnp.dot(xin, wl, preferred_element_type=_F32) + blr
    h_ids = jax.lax.broadcasted_iota(jnp.int32, (_HEADS, _C), 0)
    c_ids = jax.lax.broadcasted_iota(jnp.int32, (_HEADS, _C), 1) // _OUTC
    attmT = jnp.where(h_ids == c_ids, attr, 0.0)
    attmB = jnp.broadcast_to(attmT[None], (_DT, _HEADS, _C))
    sel = jnp.where(h_ids == c_ids, 1.0, 0.0).astype(_F32)
    alin = jax.lax.dot_general(attmT, xlT, (((1,), (0,)), ((), ())),
                               preferred_element_type=_F32)
    xre = (jnp.dot(xin[t * _DT:(t + 1) * _DT], wr,
                   preferred_element_type=_F32) + brwe)
    pairT = xlT[None, :, :] + xre[:, :, None]
    absT = jnp.abs(pairT)
    habs = jax.lax.dot_general(attmB, absT, (((2,), (1,)), ((0,), (0,))),
                               preferred_element_type=_F32)
    are = jax.lax.dot_general(xre, attmT, (((1,), (1,)), ((), ())),
                              preferred_element_type=_F32)
    neg = neg_full[t * _DT:(t + 1) * _DT]
    logits = (0.6 * (alin[None, :, :] + are[:, :, None])
              + 0.4 * habs + neg[:, None, :])
    m = jnp.max(logits, axis=2, keepdims=True)
    p = jnp.exp(logits - m)
    alpha = p / jnp.sum(p, axis=2, keepdims=True)
    agg = jnp.dot(alpha.reshape(_DT * _HEADS, _N), xl,
                  preferred_element_type=_F32).reshape(_DT, _HEADS, _C)
    out = jnp.sum(agg * sel[None], axis=1) + bias
    return jnp.where(out > 0, out, jnp.exp(out) - 1.0)


def _mega_body(x_ref,
               wl1_ref, blc1_ref, blr1_ref, wr1_ref, brwe1_ref, attr1_ref,
               bias1_ref,
               wl2_ref, blc2_ref, blr2_ref, wr2_ref, brwe2_ref, attr2_ref,
               bias2_ref, o_ref, nf_s, neg_s, h1_s, g_s):
    s = pl.program_id(0)
    is_pool = jnp.logical_or(s < _NRB,
                             jnp.logical_and(s < 3 * _NRB, s % 2 == 0))
    is_g = jnp.logical_and(jnp.logical_and(s >= _NRB, s < 5 * _NRB),
                           s % 2 == 1)
    gsub = jnp.where(s < 3 * _NRB, (s - _NRB - 1) // 2,
                     (s - 3 * _NRB - 1) // 2)
    gb = jnp.where(s < 3 * _NRB, 0, 1)

    @pl.when(is_pool)
    def _pool():
        bsel = jnp.where(s < _NRB, 0, 1)
        rb = jnp.where(s < _NRB, s, (s - _NRB) // 2)
        xb = x_ref[0]
        w_ids = jax.lax.broadcasted_iota(jnp.int32, (_W, _NW), 0) // _PS
        p_ids = jax.lax.broadcasted_iota(jnp.int32, (_W, _NW), 1)
        pmat = jnp.where(w_ids == p_ids, 1.0 / (_PS * _PS), 0.0).astype(_F32)
        for r in range(_RPP):
            srow = jnp.sum(xb[:, r * _PS:(r + 1) * _PS, :], axis=1)
            rows = jax.lax.dot_general(pmat, srow, (((0,), (1,)), ((), ())),
                                       preferred_element_type=_F32)
            nf_s[bsel, pl.ds(rb * (_RPP * _NW) + r * _NW, _NW)] = rows

    @pl.when(jnp.logical_and(is_g, gsub == 0))
    def _g0():
        nf = nf_s[gb]
        mu = jnp.mean(nf, axis=-1, keepdims=True)
        xc = nf - mu
        num = jax.lax.dot_general(xc, xc, (((1,), (1,)), ((), ())),
                                  preferred_element_type=_F32)
        nrm = jnp.sqrt(jnp.sum(xc * xc, axis=-1, keepdims=True))
        outer = jax.lax.dot_general(nrm, nrm, (((1,), (1,)), ((), ())),
                                    preferred_element_type=_F32)
        corr = num / (outer + 1e-8)
        neg_full = jnp.where(corr > _THR, 0.0, -1e30).astype(_F32)
        neg_s[...] = neg_full
        h1_s[pl.ds(0, _DT)] = _layer_tile(
            nf, neg_full, 0, wl1_ref[...], blc1_ref[...], blr1_ref[...],
            wr1_ref[...], brwe1_ref[...], attr1_ref[...], bias1_ref[...])

    @pl.when(jnp.logical_and(is_g, gsub == 1))
    def _g1():
        nf = nf_s[gb]
        h1_s[pl.ds(_DT, _DT)] = _layer_tile(
            nf, neg_s[...], 1, wl1_ref[...], blc1_ref[...], blr1_ref[...],
            wr1_ref[...], brwe1_ref[...], attr1_ref[...], bias1_ref[...])

    def _g_l2(t):
        h1 = h1_s[...]
        h2t = _layer_tile(
            h1, neg_s[...], t, wl2_ref[...], blc2_ref[...], blr2_ref[...],
            wr2_ref[...], brwe2_ref[...], attr2_ref[...], bias2_ref[...])
        for ph in range(_DT // _NW):
            g_s[gb, t * (_DT // _NW) + ph] = h2t[ph * _NW:(ph + 1) * _NW].T

    @pl.when(jnp.logical_and(is_g, gsub == 2))
    def _g2():
        _g_l2(0)

    @pl.when(jnp.logical_and(is_g, gsub == 3))
    def _g3():
        _g_l2(1)

    @pl.when(jnp.logical_and(jnp.logical_not(is_pool),
                             jnp.logical_not(is_g)))
    def _add():
        bsel = jnp.where(s < 5 * _NRB, 0, 1)
        rb = jnp.where(s < 5 * _NRB, (s - 3 * _NRB) // 2, s - 5 * _NRB)
        xb = x_ref[0]
        p_ids = jax.lax.broadcasted_iota(jnp.int32, (_NW, _W), 0)
        w_ids = jax.lax.broadcasted_iota(jnp.int32, (_NW, _W), 1) // _PS
        emat = jnp.where(p_ids == w_ids, 1.0, 0.0).astype(_F32)
        for r in range(_RPP):
            gr = g_s[bsel, pl.ds(rb * _RPP + r, 1)][0]
            wide = jnp.dot(gr, emat, preferred_element_type=_F32)
            o_ref[0, :, r * _PS:(r + 1) * _PS, :] = (
                xb[:, r * _PS:(r + 1) * _PS, :] + wide[:, None, :])


def _x_index(s):
    b = jnp.where(s < _NRB, 0,
                  jnp.where(s < 3 * _NRB, 1,
                            jnp.where(s < 5 * _NRB, 0, 1)))
    row = jnp.where(s < _NRB, s,
                    jnp.where(s < 3 * _NRB, (s - _NRB) // 2,
                              jnp.where(s < 5 * _NRB, (s - 3 * _NRB) // 2,
                                        s - 5 * _NRB)))
    return (b, 0, row, 0)


def _o_index(s):
    b = jnp.where(s < 5 * _NRB, 0, 1)
    row = jnp.where(s < 5 * _NRB,
                    jnp.maximum((s - 3 * _NRB) // 2, 0), s - 5 * _NRB)
    return (b, 0, row, 0)


def kernel(x, Wl1, bl1, Wr1, br1, We1, att1, bias1,
           Wl2, bl2, Wr2, br2, We2, att2, bias2):
    wspec = pl.BlockSpec((_C, _C), lambda s: (0, 0))
    rspec = pl.BlockSpec((1, _C), lambda s: (0, 0))
    cspec = pl.BlockSpec((_C, 1), lambda s: (0, 0))
    return pl.pallas_call(
        _mega_body,
        grid=(_NS,),
        in_specs=[
            pl.BlockSpec((1, _C, _RB, _W), _x_index),
            wspec, cspec, rspec, wspec, rspec, rspec, rspec,
            wspec, cspec, rspec, wspec, rspec, rspec, rspec,
        ],
        out_specs=pl.BlockSpec((1, _C, _RB, _W), _o_index),
        out_shape=jax.ShapeDtypeStruct((_B, _C, _H, _W), _F32),
        scratch_shapes=[
            pltpu.VMEM((_B, _N, _C), _F32),
            pltpu.VMEM((_N, _N), _F32),
            pltpu.VMEM((_N, _C), _F32),
            pltpu.VMEM((_B, _NH, _C, _NW), _F32),
        ],
    )(x,
      Wl1, bl1.reshape(_C, 1), bl1.reshape(1, _C), Wr1,
      (br1 + We1.reshape(-1)).reshape(1, _C), att1.reshape(1, _C),
      bias1.reshape(1, _C),
      Wl2, bl2.reshape(_C, 1), bl2.reshape(1, _C), Wr2,
      (br2 + We2.reshape(-1)).reshape(1, _C), att2.reshape(1, _C),
      bias2.reshape(1, _C))
